# baseline (speedup 1.0000x reference)
import functools

import numpy as np
import jax
import jax.numpy as jnp
from jax import lax
from jax.experimental import pallas as pl
from jax.experimental.pallas import tpu as pltpu
from jax.experimental.pallas import tpu_sc as plsc

_RAW_DIMS = np.array(
    [58, 58, 58, 82, 82, 82, 82, 58, 58, 58, 74, 58, 58, 58, 58, 58, 58, 74]
)
_N = 18
_P = 256
_F = 82
_E = 128
_BB = 512
_TBL = 512

_MASK = (np.arange(_F)[None, :] < _RAW_DIMS[:, None]).astype(np.float32)


def _build_adj(edge_flat):
    mesh = plsc.VectorSubcoreMesh(core_axis_name="c", subcore_axis_name="s")

    @functools.partial(
        pl.kernel,
        mesh=mesh,
        out_type=jax.ShapeDtypeStruct((_TBL,), jnp.float32),
        compiler_params=pltpu.CompilerParams(needs_layout_passes=False),
        scratch_types=[
            pltpu.VMEM((2 * _E,), jnp.int32),
            pltpu.VMEM((_E,), jnp.int32),
            pltpu.VMEM((16,), jnp.float32),
        ],
    )
    def adj_kernel(ei_hbm, out_hbm, ei_v, flat_v, acc_v):
        wid = lax.axis_index("s") * 2 + lax.axis_index("c")
        pltpu.sync_copy(ei_hbm, ei_v)
        flats = []
        for k in range(_E // 16):
            sl = pl.ds(16 * k, 16)
            flats.append(ei_v[pl.ds(_E + 16 * k, 16)] * _N + ei_v[sl])
        lanes = lax.iota(jnp.int32, 16)
        base = wid * 16
        acc = jnp.zeros((16,), jnp.int32)
        for j in range(16):
            c_splat = jnp.zeros((16,), jnp.int32) + (base + j)
            cnt = jnp.zeros((16,), jnp.int32)
            for f in flats:
                cnt = cnt + plsc.all_reduce_population_count(f == c_splat)
            acc = jnp.where(lanes == j, cnt, acc)
        acc_v[...] = acc.astype(jnp.float32)
        pltpu.sync_copy(acc_v, out_hbm.at[pl.ds(wid * 16, 16)])

    return adj_kernel(edge_flat)


def _gcn_kernel(acnt_ref, mask_ref, feats_ref, wpad_ref, bpad_ref,
                w1_ref, b1_ref, w2_ref, b2_ref, out_ref):
    f32 = jnp.float32
    eye = (jax.lax.broadcasted_iota(jnp.int32, (_N, _N), 0)
           == jax.lax.broadcasted_iota(jnp.int32, (_N, _N), 1)).astype(f32)
    a_cnt = acnt_ref[...]
    deg = jnp.sum(a_cnt, axis=1, keepdims=True) + 1.0
    dinv = jax.lax.rsqrt(deg)
    a_hat = (a_cnt + eye) * dinv * jnp.transpose(dinv)

    xm = feats_ref[...] * mask_ref[...][:, None, :]
    xs = jax.lax.dot_general(xm, wpad_ref[...],
                             (((2,), (1,)), ((0,), (0,))),
                             preferred_element_type=f32)
    xs = jnp.maximum(xs + bpad_ref[...][:, None, :], 0.0)

    h1 = jax.lax.dot_general(xs, w1_ref[...],
                             (((2,), (0,)), ((), ())),
                             preferred_element_type=f32)
    p1 = jax.lax.dot_general(a_hat, h1, (((1,), (0,)), ((), ())),
                             preferred_element_type=f32)
    x1 = jnp.maximum(p1 + b1_ref[...][None, :, :], 0.0)

    h2 = jax.lax.dot_general(x1 + xs, w2_ref[...],
                             (((2,), (0,)), ((), ())),
                             preferred_element_type=f32)
    p2 = jax.lax.dot_general(a_hat, h2, (((1,), (0,)), ((), ())),
                             preferred_element_type=f32)
    x2 = jnp.maximum(p2 + b2_ref[...][None, :, :], 0.0)

    x_out = x2 + x1
    for n in range(_N):
        out_ref[:, n, :] = x_out[n]


def kernel(feature_list_byAgentIdx, edge_index, W_pad, b_pad, W1, b1, W2, b2):
    B = feature_list_byAgentIdx.shape[1]
    mask = jnp.asarray(_MASK)
    grid = B // _BB

    a_cnt = _build_adj(edge_index.reshape(2 * _E))[:324].reshape(_N, _N)

    return pl.pallas_call(
        _gcn_kernel,
        grid=(grid,),
        in_specs=[
            pl.BlockSpec((_N, _N), lambda i: (0, 0)),
            pl.BlockSpec((_N, _F), lambda i: (0, 0)),
            pl.BlockSpec((_N, _BB, _F), lambda i: (0, i, 0)),
            pl.BlockSpec((_N, _F, _P), lambda i: (0, 0, 0)),
            pl.BlockSpec((_N, _P), lambda i: (0, 0)),
            pl.BlockSpec((_P, _P), lambda i: (0, 0)),
            pl.BlockSpec((1, _P), lambda i: (0, 0)),
            pl.BlockSpec((_P, _P), lambda i: (0, 0)),
            pl.BlockSpec((1, _P), lambda i: (0, 0)),
        ],
        out_specs=pl.BlockSpec((_BB, _N, _P), lambda i: (i, 0, 0)),
        out_shape=jax.ShapeDtypeStruct((B, _N, _P), jnp.float32),
    )(a_cnt, mask, feature_list_byAgentIdx,
      W_pad, b_pad,
      W1, b1.reshape(1, _P),
      W2, b2.reshape(1, _P))


# speedup vs baseline: 1.0457x; 1.0457x over previous
import functools

import numpy as np
import jax
import jax.numpy as jnp
from jax import lax
from jax.experimental import pallas as pl
from jax.experimental.pallas import tpu as pltpu
from jax.experimental.pallas import tpu_sc as plsc

_RAW_DIMS = np.array(
    [58, 58, 58, 82, 82, 82, 82, 58, 58, 58, 74, 58, 58, 58, 58, 58, 58, 74]
)
_N = 18
_P = 256
_F = 82
_E = 128
_BB = 256
_TBL = 32

_MASK = (np.arange(_F)[None, :] < _RAW_DIMS[:, None]).astype(np.float32)


def _build_adj(edge_flat):
    mesh = plsc.VectorSubcoreMesh(core_axis_name="c", subcore_axis_name="s")

    @functools.partial(
        pl.kernel,
        mesh=mesh,
        out_type=jax.ShapeDtypeStruct((_TBL, _TBL), jnp.float32),
        compiler_params=pltpu.CompilerParams(needs_layout_passes=False),
        scratch_types=[
            pltpu.VMEM((2 * _E,), jnp.int32),
            pltpu.VMEM((_E,), jnp.int32),
            pltpu.VMEM((16,), jnp.float32),
        ],
    )
    def adj_kernel(ei_hbm, out_hbm, ei_v, flat_v, acc_v):
        wid = lax.axis_index("s") * 2 + lax.axis_index("c")
        pltpu.sync_copy(ei_hbm, ei_v)
        flats = []
        for k in range(_E // 16):
            sl = pl.ds(16 * k, 16)
            flats.append(ei_v[pl.ds(_E + 16 * k, 16)] * _N + ei_v[sl])
        lanes = lax.iota(jnp.int32, 16)
        for chunk in (wid, wid + 32):
            row = chunk // 2
            c0 = 16 * (chunk % 2)
            acc = jnp.zeros((16,), jnp.int32)
            for j in range(16):
                c_splat = jnp.zeros((16,), jnp.int32) + (row * _N + c0 + j)
                cnt = jnp.zeros((16,), jnp.int32)
                for f in flats:
                    cnt = cnt + plsc.all_reduce_population_count(f == c_splat)
                acc = jnp.where(lanes == j, cnt, acc)
            acc_v[...] = acc.astype(jnp.float32)
            pltpu.sync_copy(acc_v, out_hbm.at[row, pl.ds(c0, 16)])

    return adj_kernel(edge_flat)


def _gcn_kernel(acnt_ref, mask_ref, feats_ref, wpad_ref, bpad_ref,
                w1_ref, b1_ref, w2_ref, b2_ref, out_ref):
    f32 = jnp.float32
    eye = (jax.lax.broadcasted_iota(jnp.int32, (_N, _N), 0)
           == jax.lax.broadcasted_iota(jnp.int32, (_N, _N), 1)).astype(f32)
    a_cnt = acnt_ref[...][:_N, :_N]
    deg = jnp.sum(a_cnt, axis=1, keepdims=True) + 1.0
    dinv = jax.lax.rsqrt(deg)
    a_hat = (a_cnt + eye) * dinv * jnp.transpose(dinv)

    xm = feats_ref[...] * mask_ref[...][:, None, :]
    xs = jax.lax.dot_general(xm, wpad_ref[...],
                             (((2,), (1,)), ((0,), (0,))),
                             preferred_element_type=f32)
    xs = jnp.maximum(xs + bpad_ref[...][:, None, :], 0.0)

    h1 = jax.lax.dot_general(xs, w1_ref[...],
                             (((2,), (0,)), ((), ())),
                             preferred_element_type=f32)
    p1 = jax.lax.dot_general(a_hat, h1, (((1,), (0,)), ((), ())),
                             preferred_element_type=f32)
    x1 = jnp.maximum(p1 + b1_ref[...][None, :, :], 0.0)

    h2 = jax.lax.dot_general(x1 + xs, w2_ref[...],
                             (((2,), (0,)), ((), ())),
                             preferred_element_type=f32)
    p2 = jax.lax.dot_general(a_hat, h2, (((1,), (0,)), ((), ())),
                             preferred_element_type=f32)
    x2 = jnp.maximum(p2 + b2_ref[...][None, :, :], 0.0)

    x_out = x2 + x1
    for n in range(_N):
        out_ref[:, n, :] = x_out[n]


def kernel(feature_list_byAgentIdx, edge_index, W_pad, b_pad, W1, b1, W2, b2):
    B = feature_list_byAgentIdx.shape[1]
    mask = jnp.asarray(_MASK)
    grid = B // _BB

    tbl = _build_adj(edge_index.reshape(2 * _E))

    return pl.pallas_call(
        _gcn_kernel,
        grid=(grid,),
        in_specs=[
            pl.BlockSpec((_TBL, _TBL), lambda i: (0, 0)),
            pl.BlockSpec((_N, _F), lambda i: (0, 0)),
            pl.BlockSpec((_N, _BB, _F), lambda i: (0, i, 0)),
            pl.BlockSpec((_N, _F, _P), lambda i: (0, 0, 0)),
            pl.BlockSpec((_N, _P), lambda i: (0, 0)),
            pl.BlockSpec((_P, _P), lambda i: (0, 0)),
            pl.BlockSpec((1, _P), lambda i: (0, 0)),
            pl.BlockSpec((_P, _P), lambda i: (0, 0)),
            pl.BlockSpec((1, _P), lambda i: (0, 0)),
        ],
        out_specs=pl.BlockSpec((_BB, _N, _P), lambda i: (i, 0, 0)),
        out_shape=jax.ShapeDtypeStruct((B, _N, _P), jnp.float32),
    )(tbl, mask, feature_list_byAgentIdx,
      W_pad, b_pad,
      W1, b1.reshape(1, _P),
      W2, b2.reshape(1, _P))
